# baseline (speedup 1.0000x reference)
import functools

import jax
import jax.numpy as jnp
from jax import lax
from jax.experimental import pallas as pl
from jax.experimental.pallas import tpu as pltpu
from jax.experimental.pallas import tpu_sc as plsc

N = 10000
E = 320000
NC = 2
NS = 16
NW = NC * NS
CH = 128
NSTEP = 79
EP = NW * NSTEP * CH
EPW = NSTEP * CH
ND = 10240
RPT = ND // NS

_mesh = functools.partial(
    plsc.VectorSubcoreMesh,
    core_axis_name="c", subcore_axis_name="s", num_cores=NC, num_subcores=NS,
)


@functools.partial(
    pl.kernel,
    out_type=jax.ShapeDtypeStruct((NC, ND), jnp.float32),
    mesh=_mesh(),
    scratch_types=[
        pltpu.VMEM((NSTEP, CH), jnp.int32),
        pltpu.VMEM((CH,), jnp.float32),
        pltpu.VMEM((ND // NS,), jnp.float32),
        pltpu.VMEM_SHARED((ND,), jnp.float32),
    ],
)
def _deg_kernel(dst_hbm, ones_hbm, zeros_hbm, out_hbm, dst_v, ones_v, strip_v, acc):
    c = lax.axis_index("c")
    s = lax.axis_index("s")
    wid = c * NS + s
    pltpu.sync_copy(zeros_hbm, strip_v)
    pltpu.sync_copy(strip_v, acc.at[pl.ds(s * (ND // NS), ND // NS)])
    pltpu.sync_copy(ones_hbm, ones_v)
    pltpu.sync_copy(dst_hbm.at[wid], dst_v)
    plsc.subcore_barrier()

    def body(j, carry):
        pltpu.sync_copy(ones_v, acc.at[dst_v.at[j]], add=True)
        return carry

    lax.fori_loop(0, NSTEP, body, 0)
    plsc.subcore_barrier()
    pltpu.sync_copy(acc.at[pl.ds(s * (ND // NS), ND // NS)], strip_v)
    pltpu.sync_copy(strip_v, out_hbm.at[c, pl.ds(s * (ND // NS), ND // NS)])


def _make_scatter(F):
    @functools.partial(
        pl.kernel,
        out_type=jax.ShapeDtypeStruct((NC, ND, F), jnp.float32),
        mesh=_mesh(),
        compiler_params=pltpu.CompilerParams(use_tc_tiling_on_sc=False),
        scratch_types=[
            pltpu.VMEM((NSTEP, CH), jnp.int32),
            pltpu.VMEM((NSTEP, CH), jnp.int32),
            pltpu.VMEM((2, CH, F), jnp.float32),
            pltpu.VMEM((RPT, F), jnp.float32),
            pltpu.VMEM_SHARED((ND, F), jnp.float32),
            pltpu.SemaphoreType.DMA((2,)),
        ],
    )
    def _scatter(g_hbm, src_hbm, dst_hbm, zeros_hbm, out_hbm,
                 src_v, dst_v, rows_v, blk_v, acc, sem):
        c = lax.axis_index("c")
        s = lax.axis_index("s")
        wid = c * NS + s
        pltpu.sync_copy(src_hbm.at[wid], src_v)
        pltpu.sync_copy(dst_hbm.at[wid], dst_v)
        pltpu.async_copy(g_hbm.at[src_v.at[0]], rows_v.at[0], sem.at[0])
        pltpu.sync_copy(zeros_hbm, blk_v)
        pltpu.sync_copy(blk_v, acc.at[pl.ds(s * RPT, RPT), :])
        plsc.subcore_barrier()

        def body(j, carry):
            p = jnp.bitwise_and(j, 1)
            pltpu.make_async_copy(g_hbm.at[pl.ds(0, CH), :], rows_v.at[p],
                                  sem.at[p]).wait()

            @pl.when(j + 1 < NSTEP)
            def _prefetch():
                pltpu.async_copy(g_hbm.at[src_v.at[j + 1]], rows_v.at[1 - p],
                                 sem.at[1 - p])

            pltpu.sync_copy(rows_v.at[p], acc.at[dst_v.at[j]], add=True)
            return carry

        lax.fori_loop(0, NSTEP, body, 0)
        plsc.subcore_barrier()
        pltpu.sync_copy(acc.at[pl.ds(s * RPT, RPT), :],
                        out_hbm.at[c, pl.ds(s * RPT, RPT), :])

    return _scatter


_scatter16 = _make_scatter(16)
_scatter24 = _make_scatter(24)
_scatter32 = _make_scatter(32)


NG = 10240
GCH = 80
GPW = NG // NW
GSTEP = GPW // GCH


@functools.partial(
    pl.kernel,
    out_type=jax.ShapeDtypeStruct((NG, 16), jnp.float32),
    mesh=_mesh(),
    compiler_params=pltpu.CompilerParams(use_tc_tiling_on_sc=False),
    scratch_types=[
        pltpu.VMEM((GSTEP, GCH), jnp.int32),
        pltpu.VMEM((GSTEP, GCH, 16), jnp.float32),
        pltpu.SemaphoreType.DMA((GSTEP,)),
        pltpu.SemaphoreType.DMA,
    ],
)
def _gather_rows(a_hbm, idx_hbm, oa_hbm, idx_v, rows_v, sem, osem):
    c = lax.axis_index("c")
    s = lax.axis_index("s")
    wid = c * NS + s
    pltpu.sync_copy(idx_hbm.at[wid], idx_v)
    for j in range(GSTEP):
        pltpu.async_copy(a_hbm.at[idx_v.at[j]], rows_v.at[j], sem.at[j])
    for j in range(GSTEP):
        pltpu.make_async_copy(a_hbm.at[pl.ds(0, GCH), :], rows_v.at[j],
                              sem.at[j]).wait()
        pltpu.async_copy(rows_v.at[j],
                         oa_hbm.at[pl.ds(wid * GPW + j * GCH, GCH), :], osem)
    for j in range(GSTEP):
        pltpu.make_async_copy(a_hbm.at[pl.ds(0, GCH), :], rows_v.at[j],
                              osem).wait()


def _tc_call(body, out_shapes):
    return pl.pallas_call(body, out_shape=out_shapes)


def _k1a_body(x_ref, w1_ref, h1_ref):
    h1_ref[...] = jnp.dot(x_ref[...], w1_ref[...],
                          preferred_element_type=jnp.float32)


def _k1b_body(h1_ref, d0_ref, d1_ref, dinv_ref, g1_ref):
    deg = d0_ref[...] + d1_ref[...] + 1.0
    dinv = 1.0 / jnp.sqrt(deg)
    dinv_ref[...] = dinv
    g1_ref[...] = h1_ref[...] * dinv


def _k_mid_body(p_ref, g_ref, dinv_ref, b_ref, w_ref, gn_ref):
    dinv = dinv_ref[...]
    h = dinv * (p_ref[0, :N, :] + p_ref[1, :N, :] + g_ref[...]) + b_ref[...]
    gn_ref[...] = jnp.dot(h, w_ref[...], preferred_element_type=jnp.float32) * dinv


def _softmax_log(z):
    m = jnp.max(z, axis=-1, keepdims=True)
    e = jnp.exp(z - m)
    p = e / jnp.sum(e, axis=-1, keepdims=True)
    return p, jnp.log(p + 1e-9)


def _k_head_body(p_ref, g_ref, dinv_ref, b_ref,
                 s1w_ref, s1b_ref, s2w_ref, s2b_ref,
                 e1w_ref, e1b_ref, e2w_ref, e2b_ref,
                 lsp_ref, ep_ref, lep_ref, el_ref):
    h = dinv_ref[...] * (p_ref[0, :N, :] + p_ref[1, :N, :] + g_ref[...]) \
        + b_ref[...]
    t = jnp.clip(jnp.dot(h, s1w_ref[...], preferred_element_type=jnp.float32)
                 + s1b_ref[...], 0.0, 6.0)
    zs = jnp.dot(t, s2w_ref[...], preferred_element_type=jnp.float32) + s2b_ref[...]
    _, lsp = _softmax_log(zs)
    lsp_ref[...] = lsp
    t2 = jnp.clip(jnp.dot(h, e1w_ref[...], preferred_element_type=jnp.float32)
                  + e1b_ref[...], 0.0, 6.0)
    ze = jnp.dot(t2, e2w_ref[...], preferred_element_type=jnp.float32) + e2b_ref[...]
    ep, lep = _softmax_log(ze)
    ep_ref[...] = ep
    lep_ref[...] = lep
    el_ref[...] = jnp.concatenate([ep, lep], axis=-1)


def kernel(x, edge_index, candidate_set, W1, b1, W2, b2, W3, b3,
           S1w, S1b, S2w, S2b, E1w, E1b, E2w, E2b):
    del candidate_set
    f32 = jnp.float32
    i32 = jnp.int32
    dst_r = jnp.concatenate(
        [edge_index[1], jnp.full((EP - E,), N + 16, i32)]).reshape(NW, NSTEP, CH)
    dst_r = jax.lax.optimization_barrier(dst_r)
    src_r = jnp.concatenate(
        [edge_index[0], jnp.zeros((EP - E,), i32)]).reshape(NW, NSTEP, CH)
    src_r = jax.lax.optimization_barrier(src_r)

    ones_ch = jnp.ones((CH,), f32)
    zeros_nd = jnp.zeros((ND // NS,), f32)
    zeros16 = jnp.zeros((RPT, 16), f32)
    zeros24 = jnp.zeros((RPT, 24), f32)
    zeros32 = jnp.zeros((RPT, 32), f32)

    gum42 = jax.random.gumbel(jax.random.key(42), (N * 8,), f32).reshape(N, 8)
    gum43 = jax.random.gumbel(
        jax.random.key(43), (2 * N * 8,), f32).reshape(2 * N, 8)
    gum42, gum43 = jax.lax.optimization_barrier((gum42, gum43))

    degp = _deg_kernel(dst_r, ones_ch, zeros_nd)
    h1 = _tc_call(_k1a_body, jax.ShapeDtypeStruct((N, 16), f32))(x, W1)
    d0 = degp[0, :N].reshape(N, 1)
    d1 = degp[1, :N].reshape(N, 1)

    dinv, g1 = _tc_call(_k1b_body, (
        jax.ShapeDtypeStruct((N, 1), f32),
        jax.ShapeDtypeStruct((N, 16), f32),
    ))(h1, d0, d1)
    p = _scatter16(g1, src_r, dst_r, zeros16)

    g2 = _tc_call(_k_mid_body, jax.ShapeDtypeStruct((N, 24), f32))(
        p, g1, dinv, b1, W2)
    p = _scatter24(g2, src_r, dst_r, zeros24)

    g3 = _tc_call(_k_mid_body, jax.ShapeDtypeStruct((N, 32), f32))(
        p, g2, dinv, b2, W3)
    p = _scatter32(g3, src_r, dst_r, zeros32)

    lsp, ep, lep, EL = _tc_call(_k_head_body, (
        jax.ShapeDtypeStruct((N, 8), f32),
        jax.ShapeDtypeStruct((N, 8), f32),
        jax.ShapeDtypeStruct((N, 8), f32),
        jax.ShapeDtypeStruct((N, 16), f32),
    ))(p, g3, dinv, b3, S1w, S1b, S2w, S2b, E1w, E1b, E2w, E2b)

    start_node = jnp.argmax(gum42 + lsp, axis=-1)

    idx = jnp.concatenate([start_node.astype(jnp.int32),
                           jnp.zeros((NG - N,), jnp.int32)]).reshape(NW, GSTEP, GCH)
    en1 = jnp.argmax(gum43[:N] + lep, axis=-1)
    ELs = _gather_rows(EL, idx)

    end_probs = jnp.concatenate([ep, ELs[:N, :8]], axis=0)
    en2 = jnp.argmax(gum43[N:] + ELs[:N, 8:], axis=-1)
    end_node = jnp.concatenate([en1, en2], axis=0)
    return (start_node, end_node, end_probs)


# speedup vs baseline: 1.1952x; 1.1952x over previous
import functools

import jax
import jax.numpy as jnp
from jax import lax
from jax.experimental import pallas as pl
from jax.experimental.pallas import tpu as pltpu
from jax.experimental.pallas import tpu_sc as plsc

N = 10000
E = 320000
NC = 2
NS = 16
NW = NC * NS
CH = 128
NSTEP = 79
EP = NW * NSTEP * CH
EPW = NSTEP * CH
ND = 10240
RPT = ND // NS

_mesh = functools.partial(
    plsc.VectorSubcoreMesh,
    core_axis_name="c", subcore_axis_name="s", num_cores=NC, num_subcores=NS,
)


@functools.partial(
    pl.kernel,
    out_type=jax.ShapeDtypeStruct((NC, ND), jnp.float32),
    mesh=_mesh(),
    scratch_types=[
        pltpu.VMEM((NSTEP, CH), jnp.int32),
        pltpu.VMEM((CH,), jnp.float32),
        pltpu.VMEM((ND // NS,), jnp.float32),
        pltpu.VMEM_SHARED((ND,), jnp.float32),
    ],
)
def _deg_kernel(dst_hbm, ones_hbm, zeros_hbm, out_hbm, dst_v, ones_v, strip_v, acc):
    c = lax.axis_index("c")
    s = lax.axis_index("s")
    wid = c * NS + s
    pltpu.sync_copy(zeros_hbm, strip_v)
    pltpu.sync_copy(strip_v, acc.at[pl.ds(s * (ND // NS), ND // NS)])
    pltpu.sync_copy(ones_hbm, ones_v)
    pltpu.sync_copy(dst_hbm.at[wid], dst_v)
    plsc.subcore_barrier()

    def body(j, carry):
        pltpu.sync_copy(ones_v, acc.at[dst_v.at[j]], add=True)
        return carry

    lax.fori_loop(0, NSTEP, body, 0)
    plsc.subcore_barrier()
    pltpu.sync_copy(acc.at[pl.ds(s * (ND // NS), ND // NS)], strip_v)
    pltpu.sync_copy(strip_v, out_hbm.at[c, pl.ds(s * (ND // NS), ND // NS)])


def _make_scatter(F):
    @functools.partial(
        pl.kernel,
        out_type=jax.ShapeDtypeStruct((NC, ND, F), jnp.float32),
        mesh=_mesh(),
        compiler_params=pltpu.CompilerParams(use_tc_tiling_on_sc=False),
        scratch_types=[
            pltpu.VMEM((NSTEP, CH), jnp.int32),
            pltpu.VMEM((NSTEP, CH), jnp.int32),
            pltpu.VMEM((4, CH, F), jnp.float32),
            pltpu.VMEM((RPT, F), jnp.float32),
            pltpu.VMEM_SHARED((ND, F), jnp.float32),
            pltpu.SemaphoreType.DMA((4,)),
            pltpu.SemaphoreType.DMA((4,)),
        ],
    )
    def _scatter(g_hbm, src_hbm, dst_hbm, zeros_hbm, out_hbm,
                 src_v, dst_v, rows_v, blk_v, acc, gsem, ssem):
        c = lax.axis_index("c")
        s = lax.axis_index("s")
        wid = c * NS + s
        pltpu.sync_copy(src_hbm.at[wid], src_v)
        pltpu.sync_copy(dst_hbm.at[wid], dst_v)
        pltpu.async_copy(g_hbm.at[src_v.at[0]], rows_v.at[0], gsem.at[0])
        pltpu.async_copy(g_hbm.at[src_v.at[1]], rows_v.at[1], gsem.at[1])
        pltpu.sync_copy(zeros_hbm, blk_v)
        pltpu.sync_copy(blk_v, acc.at[pl.ds(s * RPT, RPT), :])
        plsc.subcore_barrier()

        def body(j, carry):
            p = jnp.bitwise_and(j, 3)
            pltpu.make_async_copy(g_hbm.at[pl.ds(0, CH), :], rows_v.at[p],
                                  gsem.at[p]).wait()
            pltpu.async_copy(rows_v.at[p], acc.at[dst_v.at[j]], ssem.at[p],
                             add=True)

            @pl.when(j + 2 < NSTEP)
            def _prefetch():
                q = jnp.bitwise_and(j + 2, 3)

                @pl.when(j >= 2)
                def _reuse():
                    pltpu.make_async_copy(rows_v.at[q],
                                          acc.at[pl.ds(0, CH), :],
                                          ssem.at[q]).wait()

                pltpu.async_copy(g_hbm.at[src_v.at[j + 2]], rows_v.at[q],
                                 gsem.at[q])
            return carry

        lax.fori_loop(0, NSTEP, body, 0)
        for t in (NSTEP - 4, NSTEP - 3, NSTEP - 2, NSTEP - 1):
            pltpu.make_async_copy(rows_v.at[t & 3], acc.at[pl.ds(0, CH), :],
                                  ssem.at[t & 3]).wait()
        plsc.subcore_barrier()
        pltpu.sync_copy(acc.at[pl.ds(s * RPT, RPT), :],
                        out_hbm.at[c, pl.ds(s * RPT, RPT), :])

    return _scatter


_scatter16 = _make_scatter(16)
_scatter24 = _make_scatter(24)
_scatter32 = _make_scatter(32)


NG = 10240
GCH = 80
GPW = NG // NW
GSTEP = GPW // GCH


@functools.partial(
    pl.kernel,
    out_type=jax.ShapeDtypeStruct((NG, 16), jnp.float32),
    mesh=_mesh(),
    compiler_params=pltpu.CompilerParams(use_tc_tiling_on_sc=False),
    scratch_types=[
        pltpu.VMEM((GSTEP, GCH), jnp.int32),
        pltpu.VMEM((GSTEP, GCH, 16), jnp.float32),
        pltpu.SemaphoreType.DMA((GSTEP,)),
        pltpu.SemaphoreType.DMA,
    ],
)
def _gather_rows(a_hbm, idx_hbm, oa_hbm, idx_v, rows_v, sem, osem):
    c = lax.axis_index("c")
    s = lax.axis_index("s")
    wid = c * NS + s
    pltpu.sync_copy(idx_hbm.at[wid], idx_v)
    for j in range(GSTEP):
        pltpu.async_copy(a_hbm.at[idx_v.at[j]], rows_v.at[j], sem.at[j])
    for j in range(GSTEP):
        pltpu.make_async_copy(a_hbm.at[pl.ds(0, GCH), :], rows_v.at[j],
                              sem.at[j]).wait()
        pltpu.async_copy(rows_v.at[j],
                         oa_hbm.at[pl.ds(wid * GPW + j * GCH, GCH), :], osem)
    for j in range(GSTEP):
        pltpu.make_async_copy(a_hbm.at[pl.ds(0, GCH), :], rows_v.at[j],
                              osem).wait()


def _tc_call(body, out_shapes):
    return pl.pallas_call(body, out_shape=out_shapes)


def _k1a_body(x_ref, w1_ref, h1_ref):
    h1_ref[...] = jnp.dot(x_ref[...], w1_ref[...],
                          preferred_element_type=jnp.float32)


def _k1b_body(h1_ref, d0_ref, d1_ref, dinv_ref, g1_ref):
    deg = d0_ref[...] + d1_ref[...] + 1.0
    dinv = 1.0 / jnp.sqrt(deg)
    dinv_ref[...] = dinv
    g1_ref[...] = h1_ref[...] * dinv


def _k_mid_body(p_ref, g_ref, dinv_ref, b_ref, w_ref, gn_ref):
    dinv = dinv_ref[...]
    h = dinv * (p_ref[0, :N, :] + p_ref[1, :N, :] + g_ref[...]) + b_ref[...]
    gn_ref[...] = jnp.dot(h, w_ref[...], preferred_element_type=jnp.float32) * dinv


def _softmax_log(z):
    m = jnp.max(z, axis=-1, keepdims=True)
    e = jnp.exp(z - m)
    p = e / jnp.sum(e, axis=-1, keepdims=True)
    return p, jnp.log(p + 1e-9)


def _k_head_body(p_ref, g_ref, dinv_ref, b_ref,
                 s1w_ref, s1b_ref, s2w_ref, s2b_ref,
                 e1w_ref, e1b_ref, e2w_ref, e2b_ref,
                 lsp_ref, ep_ref, lep_ref, el_ref):
    h = dinv_ref[...] * (p_ref[0, :N, :] + p_ref[1, :N, :] + g_ref[...]) \
        + b_ref[...]
    t = jnp.clip(jnp.dot(h, s1w_ref[...], preferred_element_type=jnp.float32)
                 + s1b_ref[...], 0.0, 6.0)
    zs = jnp.dot(t, s2w_ref[...], preferred_element_type=jnp.float32) + s2b_ref[...]
    _, lsp = _softmax_log(zs)
    lsp_ref[...] = lsp
    t2 = jnp.clip(jnp.dot(h, e1w_ref[...], preferred_element_type=jnp.float32)
                  + e1b_ref[...], 0.0, 6.0)
    ze = jnp.dot(t2, e2w_ref[...], preferred_element_type=jnp.float32) + e2b_ref[...]
    ep, lep = _softmax_log(ze)
    ep_ref[...] = ep
    lep_ref[...] = lep
    el_ref[...] = jnp.concatenate([ep, lep], axis=-1)


def kernel(x, edge_index, candidate_set, W1, b1, W2, b2, W3, b3,
           S1w, S1b, S2w, S2b, E1w, E1b, E2w, E2b):
    del candidate_set
    f32 = jnp.float32
    i32 = jnp.int32
    dst_r = jnp.concatenate(
        [edge_index[1], jnp.full((EP - E,), N + 16, i32)]).reshape(NW, NSTEP, CH)
    dst_r = jax.lax.optimization_barrier(dst_r)
    src_r = jnp.concatenate(
        [edge_index[0], jnp.zeros((EP - E,), i32)]).reshape(NW, NSTEP, CH)
    src_r = jax.lax.optimization_barrier(src_r)

    ones_ch = jnp.ones((CH,), f32)
    zeros_nd = jnp.zeros((ND // NS,), f32)
    zeros16 = jnp.zeros((RPT, 16), f32)
    zeros24 = jnp.zeros((RPT, 24), f32)
    zeros32 = jnp.zeros((RPT, 32), f32)

    gum42 = jax.random.gumbel(jax.random.key(42), (N * 8,), f32).reshape(N, 8)
    gum43 = jax.random.gumbel(
        jax.random.key(43), (2 * N * 8,), f32).reshape(2 * N, 8)
    gum42, gum43 = jax.lax.optimization_barrier((gum42, gum43))

    degp = _deg_kernel(dst_r, ones_ch, zeros_nd)
    h1 = _tc_call(_k1a_body, jax.ShapeDtypeStruct((N, 16), f32))(x, W1)
    d0 = degp[0, :N].reshape(N, 1)
    d1 = degp[1, :N].reshape(N, 1)

    dinv, g1 = _tc_call(_k1b_body, (
        jax.ShapeDtypeStruct((N, 1), f32),
        jax.ShapeDtypeStruct((N, 16), f32),
    ))(h1, d0, d1)
    p = _scatter16(g1, src_r, dst_r, zeros16)

    g2 = _tc_call(_k_mid_body, jax.ShapeDtypeStruct((N, 24), f32))(
        p, g1, dinv, b1, W2)
    p = _scatter24(g2, src_r, dst_r, zeros24)

    g3 = _tc_call(_k_mid_body, jax.ShapeDtypeStruct((N, 32), f32))(
        p, g2, dinv, b2, W3)
    p = _scatter32(g3, src_r, dst_r, zeros32)

    lsp, ep, lep, EL = _tc_call(_k_head_body, (
        jax.ShapeDtypeStruct((N, 8), f32),
        jax.ShapeDtypeStruct((N, 8), f32),
        jax.ShapeDtypeStruct((N, 8), f32),
        jax.ShapeDtypeStruct((N, 16), f32),
    ))(p, g3, dinv, b3, S1w, S1b, S2w, S2b, E1w, E1b, E2w, E2b)

    start_node = jnp.argmax(gum42 + lsp, axis=-1)

    idx = jnp.concatenate([start_node.astype(jnp.int32),
                           jnp.zeros((NG - N,), jnp.int32)]).reshape(NW, GSTEP, GCH)
    en1 = jnp.argmax(gum43[:N] + lep, axis=-1)
    ELs = _gather_rows(EL, idx)

    end_probs = jnp.concatenate([ep, ELs[:N, :8]], axis=0)
    en2 = jnp.argmax(gum43[N:] + ELs[:N, 8:], axis=-1)
    end_node = jnp.concatenate([en1, en2], axis=0)
    return (start_node, end_node, end_probs)


# speedup vs baseline: 1.2533x; 1.0486x over previous
import functools

import jax
import jax.numpy as jnp
from jax import lax
from jax.experimental import pallas as pl
from jax.experimental.pallas import tpu as pltpu
from jax.experimental.pallas import tpu_sc as plsc

N = 10000
E = 320000
NC = 2
NS = 16
NW = NC * NS
CH = 128
NSTEP = 79
EP = NW * NSTEP * CH
EPW = NSTEP * CH
ND = 10240
RPT = ND // NS

_mesh = functools.partial(
    plsc.VectorSubcoreMesh,
    core_axis_name="c", subcore_axis_name="s", num_cores=NC, num_subcores=NS,
)


@functools.partial(
    pl.kernel,
    out_type=jax.ShapeDtypeStruct((NC, ND), jnp.float32),
    mesh=_mesh(),
    scratch_types=[
        pltpu.VMEM((NSTEP, CH), jnp.int32),
        pltpu.VMEM((CH,), jnp.float32),
        pltpu.VMEM((ND // NS,), jnp.float32),
        pltpu.VMEM_SHARED((ND,), jnp.float32),
    ],
)
def _deg_kernel(dst_hbm, ones_hbm, zeros_hbm, out_hbm, dst_v, ones_v, strip_v, acc):
    c = lax.axis_index("c")
    s = lax.axis_index("s")
    wid = c * NS + s
    pltpu.sync_copy(zeros_hbm, strip_v)
    pltpu.sync_copy(strip_v, acc.at[pl.ds(s * (ND // NS), ND // NS)])
    pltpu.sync_copy(ones_hbm, ones_v)
    pltpu.sync_copy(dst_hbm.at[wid], dst_v)
    plsc.subcore_barrier()

    def body(j, carry):
        pltpu.sync_copy(ones_v, acc.at[dst_v.at[j]], add=True)
        return carry

    lax.fori_loop(0, NSTEP, body, 0)
    plsc.subcore_barrier()
    pltpu.sync_copy(acc.at[pl.ds(s * (ND // NS), ND // NS)], strip_v)
    pltpu.sync_copy(strip_v, out_hbm.at[c, pl.ds(s * (ND // NS), ND // NS)])


def _make_scatter(F):
    @functools.partial(
        pl.kernel,
        out_type=jax.ShapeDtypeStruct((NC, ND, F), jnp.float32),
        mesh=_mesh(),
        compiler_params=pltpu.CompilerParams(use_tc_tiling_on_sc=False),
        scratch_types=[
            pltpu.VMEM((NSTEP, CH), jnp.int32),
            pltpu.VMEM((NSTEP, CH), jnp.int32),
            pltpu.VMEM((8, CH, F), jnp.float32),
            pltpu.VMEM((RPT, F), jnp.float32),
            pltpu.VMEM_SHARED((ND, F), jnp.float32),
            pltpu.SemaphoreType.DMA((8,)),
            pltpu.SemaphoreType.DMA((8,)),
        ],
    )
    def _scatter(g_hbm, src_hbm, dst_hbm, zeros_hbm, out_hbm,
                 src_v, dst_v, rows_v, blk_v, acc, gsem, ssem):
        c = lax.axis_index("c")
        s = lax.axis_index("s")
        wid = c * NS + s
        pltpu.sync_copy(src_hbm.at[wid], src_v)
        pltpu.sync_copy(dst_hbm.at[wid], dst_v)
        for t in (0, 1, 2, 3):
            pltpu.async_copy(g_hbm.at[src_v.at[t]], rows_v.at[t], gsem.at[t])
        pltpu.sync_copy(zeros_hbm, blk_v)
        pltpu.sync_copy(blk_v, acc.at[pl.ds(s * RPT, RPT), :])
        plsc.subcore_barrier()

        def body(j, carry):
            p = jnp.bitwise_and(j, 7)
            pltpu.make_async_copy(g_hbm.at[pl.ds(0, CH), :], rows_v.at[p],
                                  gsem.at[p]).wait()
            pltpu.async_copy(rows_v.at[p], acc.at[dst_v.at[j]], ssem.at[p],
                             add=True)

            @pl.when(j + 4 < NSTEP)
            def _prefetch():
                q = jnp.bitwise_and(j + 4, 7)

                @pl.when(j >= 4)
                def _reuse():
                    pltpu.make_async_copy(rows_v.at[q],
                                          acc.at[pl.ds(0, CH), :],
                                          ssem.at[q]).wait()

                pltpu.async_copy(g_hbm.at[src_v.at[j + 4]], rows_v.at[q],
                                 gsem.at[q])
            return carry

        lax.fori_loop(0, NSTEP, body, 0)
        for t in range(NSTEP - 8, NSTEP):
            pltpu.make_async_copy(rows_v.at[t & 7], acc.at[pl.ds(0, CH), :],
                                  ssem.at[t & 7]).wait()
        plsc.subcore_barrier()
        pltpu.sync_copy(acc.at[pl.ds(s * RPT, RPT), :],
                        out_hbm.at[c, pl.ds(s * RPT, RPT), :])

    return _scatter


_scatter16 = _make_scatter(16)
_scatter24 = _make_scatter(24)
_scatter32 = _make_scatter(32)


NG = 10240
GCH = 80
GPW = NG // NW
GSTEP = GPW // GCH


@functools.partial(
    pl.kernel,
    out_type=jax.ShapeDtypeStruct((NG, 16), jnp.float32),
    mesh=_mesh(),
    compiler_params=pltpu.CompilerParams(use_tc_tiling_on_sc=False),
    scratch_types=[
        pltpu.VMEM((GSTEP, GCH), jnp.int32),
        pltpu.VMEM((GSTEP, GCH, 16), jnp.float32),
        pltpu.SemaphoreType.DMA((GSTEP,)),
        pltpu.SemaphoreType.DMA,
    ],
)
def _gather_rows(a_hbm, idx_hbm, oa_hbm, idx_v, rows_v, sem, osem):
    c = lax.axis_index("c")
    s = lax.axis_index("s")
    wid = c * NS + s
    pltpu.sync_copy(idx_hbm.at[wid], idx_v)
    for j in range(GSTEP):
        pltpu.async_copy(a_hbm.at[idx_v.at[j]], rows_v.at[j], sem.at[j])
    for j in range(GSTEP):
        pltpu.make_async_copy(a_hbm.at[pl.ds(0, GCH), :], rows_v.at[j],
                              sem.at[j]).wait()
        pltpu.async_copy(rows_v.at[j],
                         oa_hbm.at[pl.ds(wid * GPW + j * GCH, GCH), :], osem)
    for j in range(GSTEP):
        pltpu.make_async_copy(a_hbm.at[pl.ds(0, GCH), :], rows_v.at[j],
                              osem).wait()


def _tc_call(body, out_shapes):
    return pl.pallas_call(body, out_shape=out_shapes)


def _k1a_body(x_ref, w1_ref, h1_ref):
    h1_ref[...] = jnp.dot(x_ref[...], w1_ref[...],
                          preferred_element_type=jnp.float32)


def _k1b_body(h1_ref, d0_ref, d1_ref, dinv_ref, g1_ref):
    deg = d0_ref[...] + d1_ref[...] + 1.0
    dinv = 1.0 / jnp.sqrt(deg)
    dinv_ref[...] = dinv
    g1_ref[...] = h1_ref[...] * dinv


def _k_mid_body(p_ref, g_ref, dinv_ref, b_ref, w_ref, gn_ref):
    dinv = dinv_ref[...]
    h = dinv * (p_ref[0, :N, :] + p_ref[1, :N, :] + g_ref[...]) + b_ref[...]
    gn_ref[...] = jnp.dot(h, w_ref[...], preferred_element_type=jnp.float32) * dinv


def _softmax_log(z):
    m = jnp.max(z, axis=-1, keepdims=True)
    e = jnp.exp(z - m)
    p = e / jnp.sum(e, axis=-1, keepdims=True)
    return p, jnp.log(p + 1e-9)


def _k_head_body(p_ref, g_ref, dinv_ref, b_ref,
                 s1w_ref, s1b_ref, s2w_ref, s2b_ref,
                 e1w_ref, e1b_ref, e2w_ref, e2b_ref,
                 lsp_ref, ep_ref, lep_ref, el_ref):
    h = dinv_ref[...] * (p_ref[0, :N, :] + p_ref[1, :N, :] + g_ref[...]) \
        + b_ref[...]
    t = jnp.clip(jnp.dot(h, s1w_ref[...], preferred_element_type=jnp.float32)
                 + s1b_ref[...], 0.0, 6.0)
    zs = jnp.dot(t, s2w_ref[...], preferred_element_type=jnp.float32) + s2b_ref[...]
    _, lsp = _softmax_log(zs)
    lsp_ref[...] = lsp
    t2 = jnp.clip(jnp.dot(h, e1w_ref[...], preferred_element_type=jnp.float32)
                  + e1b_ref[...], 0.0, 6.0)
    ze = jnp.dot(t2, e2w_ref[...], preferred_element_type=jnp.float32) + e2b_ref[...]
    ep, lep = _softmax_log(ze)
    ep_ref[...] = ep
    lep_ref[...] = lep
    el_ref[...] = jnp.concatenate([ep, lep], axis=-1)


def kernel(x, edge_index, candidate_set, W1, b1, W2, b2, W3, b3,
           S1w, S1b, S2w, S2b, E1w, E1b, E2w, E2b):
    del candidate_set
    f32 = jnp.float32
    i32 = jnp.int32
    dst_r = jnp.concatenate(
        [edge_index[1], jnp.full((EP - E,), N + 16, i32)]).reshape(NW, NSTEP, CH)
    dst_r = jax.lax.optimization_barrier(dst_r)
    src_r = jnp.concatenate(
        [edge_index[0], jnp.zeros((EP - E,), i32)]).reshape(NW, NSTEP, CH)
    src_r = jax.lax.optimization_barrier(src_r)

    ones_ch = jnp.ones((CH,), f32)
    zeros_nd = jnp.zeros((ND // NS,), f32)
    zeros16 = jnp.zeros((RPT, 16), f32)
    zeros24 = jnp.zeros((RPT, 24), f32)
    zeros32 = jnp.zeros((RPT, 32), f32)

    gum42 = jax.random.gumbel(jax.random.key(42), (N * 8,), f32).reshape(N, 8)
    gum43 = jax.random.gumbel(
        jax.random.key(43), (2 * N * 8,), f32).reshape(2 * N, 8)
    gum42, gum43 = jax.lax.optimization_barrier((gum42, gum43))

    degp = _deg_kernel(dst_r, ones_ch, zeros_nd)
    h1 = _tc_call(_k1a_body, jax.ShapeDtypeStruct((N, 16), f32))(x, W1)
    d0 = degp[0, :N].reshape(N, 1)
    d1 = degp[1, :N].reshape(N, 1)

    dinv, g1 = _tc_call(_k1b_body, (
        jax.ShapeDtypeStruct((N, 1), f32),
        jax.ShapeDtypeStruct((N, 16), f32),
    ))(h1, d0, d1)
    p = _scatter16(g1, src_r, dst_r, zeros16)

    g2 = _tc_call(_k_mid_body, jax.ShapeDtypeStruct((N, 24), f32))(
        p, g1, dinv, b1, W2)
    p = _scatter24(g2, src_r, dst_r, zeros24)

    g3 = _tc_call(_k_mid_body, jax.ShapeDtypeStruct((N, 32), f32))(
        p, g2, dinv, b2, W3)
    p = _scatter32(g3, src_r, dst_r, zeros32)

    lsp, ep, lep, EL = _tc_call(_k_head_body, (
        jax.ShapeDtypeStruct((N, 8), f32),
        jax.ShapeDtypeStruct((N, 8), f32),
        jax.ShapeDtypeStruct((N, 8), f32),
        jax.ShapeDtypeStruct((N, 16), f32),
    ))(p, g3, dinv, b3, S1w, S1b, S2w, S2b, E1w, E1b, E2w, E2b)

    start_node = jnp.argmax(gum42 + lsp, axis=-1)

    idx = jnp.concatenate([start_node.astype(jnp.int32),
                           jnp.zeros((NG - N,), jnp.int32)]).reshape(NW, GSTEP, GCH)
    en1 = jnp.argmax(gum43[:N] + lep, axis=-1)
    ELs = _gather_rows(EL, idx)

    end_probs = jnp.concatenate([ep, ELs[:N, :8]], axis=0)
    en2 = jnp.argmax(gum43[N:] + ELs[:N, 8:], axis=-1)
    end_node = jnp.concatenate([en1, en2], axis=0)
    return (start_node, end_node, end_probs)


# speedup vs baseline: 1.2533x; 1.0000x over previous
import functools

import jax
import jax.numpy as jnp
from jax import lax
from jax.experimental import pallas as pl
from jax.experimental.pallas import tpu as pltpu
from jax.experimental.pallas import tpu_sc as plsc

N = 10000
E = 320000
NC = 2
NS = 16
NW = NC * NS
CH = 128
NSTEP = 79
EP = NW * NSTEP * CH
EPW = NSTEP * CH
ND = 10240
RPT = ND // NS

_mesh = functools.partial(
    plsc.VectorSubcoreMesh,
    core_axis_name="c", subcore_axis_name="s", num_cores=NC, num_subcores=NS,
)


@functools.partial(
    pl.kernel,
    out_type=jax.ShapeDtypeStruct((NC, ND), jnp.float32),
    mesh=_mesh(),
    scratch_types=[
        pltpu.VMEM((NSTEP, CH), jnp.int32),
        pltpu.VMEM((CH,), jnp.float32),
        pltpu.VMEM((ND // NS,), jnp.float32),
        pltpu.VMEM_SHARED((ND,), jnp.float32),
    ],
)
def _deg_kernel(dst_hbm, ones_hbm, zeros_hbm, out_hbm, dst_v, ones_v, strip_v, acc):
    c = lax.axis_index("c")
    s = lax.axis_index("s")
    wid = c * NS + s
    pltpu.sync_copy(zeros_hbm, strip_v)
    pltpu.sync_copy(strip_v, acc.at[pl.ds(s * (ND // NS), ND // NS)])
    pltpu.sync_copy(ones_hbm, ones_v)
    pltpu.sync_copy(dst_hbm.at[wid], dst_v)
    plsc.subcore_barrier()

    def body(j, carry):
        pltpu.sync_copy(ones_v, acc.at[dst_v.at[j]], add=True)
        return carry

    lax.fori_loop(0, NSTEP, body, 0)
    plsc.subcore_barrier()
    pltpu.sync_copy(acc.at[pl.ds(s * (ND // NS), ND // NS)], strip_v)
    pltpu.sync_copy(strip_v, out_hbm.at[c, pl.ds(s * (ND // NS), ND // NS)])


def _make_scatter(F):
    @functools.partial(
        pl.kernel,
        out_type=jax.ShapeDtypeStruct((NC, ND, F), jnp.float32),
        mesh=_mesh(),
        compiler_params=pltpu.CompilerParams(use_tc_tiling_on_sc=False),
        scratch_types=[
            pltpu.VMEM((NSTEP, CH), jnp.int32),
            pltpu.VMEM((NSTEP, CH), jnp.int32),
            pltpu.VMEM((8, CH, F), jnp.float32),
            pltpu.VMEM((RPT, F), jnp.float32),
            pltpu.VMEM_SHARED((ND, F), jnp.float32),
            pltpu.SemaphoreType.DMA((8,)),
            pltpu.SemaphoreType.DMA((8,)),
        ],
    )
    def _scatter(g_hbm, src_hbm, dst_hbm, zeros_hbm, out_hbm,
                 src_v, dst_v, rows_v, blk_v, acc, gsem, ssem):
        c = lax.axis_index("c")
        s = lax.axis_index("s")
        wid = c * NS + s
        pltpu.sync_copy(src_hbm.at[wid], src_v)
        pltpu.sync_copy(dst_hbm.at[wid], dst_v)
        for t in range(4):
            pltpu.async_copy(g_hbm.at[src_v.at[t]], rows_v.at[t], gsem.at[t])
        pltpu.sync_copy(zeros_hbm, blk_v)
        pltpu.sync_copy(blk_v, acc.at[pl.ds(s * RPT, RPT), :])
        plsc.subcore_barrier()

        def body(j, carry):
            p = jnp.bitwise_and(j, 7)
            pltpu.make_async_copy(g_hbm.at[pl.ds(0, CH), :], rows_v.at[p],
                                  gsem.at[p]).wait()
            pltpu.async_copy(rows_v.at[p], acc.at[dst_v.at[j]], ssem.at[p],
                             add=True)

            @pl.when(j + 4 < NSTEP)
            def _prefetch():
                q = jnp.bitwise_and(j + 4, 7)

                @pl.when(j >= 4)
                def _reuse():
                    pltpu.make_async_copy(rows_v.at[q],
                                          acc.at[pl.ds(0, CH), :],
                                          ssem.at[q]).wait()

                pltpu.async_copy(g_hbm.at[src_v.at[j + 4]], rows_v.at[q],
                                 gsem.at[q])
            return carry

        lax.fori_loop(0, NSTEP, body, 0)
        for t in range(NSTEP - 8, NSTEP):
            pltpu.make_async_copy(rows_v.at[t & 7], acc.at[pl.ds(0, CH), :],
                                  ssem.at[t & 7]).wait()
        plsc.subcore_barrier()
        pltpu.sync_copy(acc.at[pl.ds(s * RPT, RPT), :],
                        out_hbm.at[c, pl.ds(s * RPT, RPT), :])

    return _scatter


_scatter16 = _make_scatter(16)
_scatter24 = _make_scatter(24)
_scatter32 = _make_scatter(32)


NG = 10240
GCH = 80
GPW = NG // NW
GSTEP = GPW // GCH


@functools.partial(
    pl.kernel,
    out_type=jax.ShapeDtypeStruct((NG, 16), jnp.float32),
    mesh=_mesh(),
    compiler_params=pltpu.CompilerParams(use_tc_tiling_on_sc=False),
    scratch_types=[
        pltpu.VMEM((GSTEP, GCH), jnp.int32),
        pltpu.VMEM((GSTEP, GCH, 16), jnp.float32),
        pltpu.SemaphoreType.DMA((GSTEP,)),
        pltpu.SemaphoreType.DMA,
    ],
)
def _gather_rows(a_hbm, idx_hbm, oa_hbm, idx_v, rows_v, sem, osem):
    c = lax.axis_index("c")
    s = lax.axis_index("s")
    wid = c * NS + s
    pltpu.sync_copy(idx_hbm.at[wid], idx_v)
    for j in range(GSTEP):
        pltpu.async_copy(a_hbm.at[idx_v.at[j]], rows_v.at[j], sem.at[j])
    for j in range(GSTEP):
        pltpu.make_async_copy(a_hbm.at[pl.ds(0, GCH), :], rows_v.at[j],
                              sem.at[j]).wait()
        pltpu.async_copy(rows_v.at[j],
                         oa_hbm.at[pl.ds(wid * GPW + j * GCH, GCH), :], osem)
    for j in range(GSTEP):
        pltpu.make_async_copy(a_hbm.at[pl.ds(0, GCH), :], rows_v.at[j],
                              osem).wait()


def _tc_call(body, out_shapes):
    return pl.pallas_call(body, out_shape=out_shapes)


def _k1a_body(x_ref, w1_ref, h1_ref):
    h1_ref[...] = jnp.dot(x_ref[...], w1_ref[...],
                          preferred_element_type=jnp.float32)


def _k1b_body(h1_ref, d0_ref, d1_ref, dinv_ref, g1_ref):
    deg = d0_ref[...] + d1_ref[...] + 1.0
    dinv = 1.0 / jnp.sqrt(deg)
    dinv_ref[...] = dinv
    g1_ref[...] = h1_ref[...] * dinv


def _k_mid_body(p_ref, g_ref, dinv_ref, b_ref, w_ref, gn_ref):
    dinv = dinv_ref[...]
    h = dinv * (p_ref[0, :N, :] + p_ref[1, :N, :] + g_ref[...]) + b_ref[...]
    gn_ref[...] = jnp.dot(h, w_ref[...], preferred_element_type=jnp.float32) * dinv


def _softmax_log(z):
    m = jnp.max(z, axis=-1, keepdims=True)
    e = jnp.exp(z - m)
    p = e / jnp.sum(e, axis=-1, keepdims=True)
    return p, jnp.log(p + 1e-9)


def _k_head_body(p_ref, g_ref, dinv_ref, b_ref,
                 s1w_ref, s1b_ref, s2w_ref, s2b_ref,
                 e1w_ref, e1b_ref, e2w_ref, e2b_ref,
                 lsp_ref, ep_ref, lep_ref, el_ref):
    h = dinv_ref[...] * (p_ref[0, :N, :] + p_ref[1, :N, :] + g_ref[...]) \
        + b_ref[...]
    t = jnp.clip(jnp.dot(h, s1w_ref[...], preferred_element_type=jnp.float32)
                 + s1b_ref[...], 0.0, 6.0)
    zs = jnp.dot(t, s2w_ref[...], preferred_element_type=jnp.float32) + s2b_ref[...]
    _, lsp = _softmax_log(zs)
    lsp_ref[...] = lsp
    t2 = jnp.clip(jnp.dot(h, e1w_ref[...], preferred_element_type=jnp.float32)
                  + e1b_ref[...], 0.0, 6.0)
    ze = jnp.dot(t2, e2w_ref[...], preferred_element_type=jnp.float32) + e2b_ref[...]
    ep, lep = _softmax_log(ze)
    ep_ref[...] = ep
    lep_ref[...] = lep
    el_ref[...] = jnp.concatenate([ep, lep], axis=-1)


def kernel(x, edge_index, candidate_set, W1, b1, W2, b2, W3, b3,
           S1w, S1b, S2w, S2b, E1w, E1b, E2w, E2b):
    del candidate_set
    f32 = jnp.float32
    i32 = jnp.int32
    dst_r = jnp.concatenate(
        [edge_index[1], jnp.full((EP - E,), N + 16, i32)]).reshape(NW, NSTEP, CH)
    dst_r = jax.lax.optimization_barrier(dst_r)
    src_r = jnp.concatenate(
        [edge_index[0], jnp.zeros((EP - E,), i32)]).reshape(NW, NSTEP, CH)
    src_r = jax.lax.optimization_barrier(src_r)

    ones_ch = jnp.ones((CH,), f32)
    zeros_nd = jnp.zeros((ND // NS,), f32)
    zeros16 = jnp.zeros((RPT, 16), f32)
    zeros24 = jnp.zeros((RPT, 24), f32)
    zeros32 = jnp.zeros((RPT, 32), f32)

    gum42 = jax.random.gumbel(jax.random.key(42), (N * 8,), f32).reshape(N, 8)
    gum43 = jax.random.gumbel(
        jax.random.key(43), (2 * N * 8,), f32).reshape(2 * N, 8)
    gum42, gum43 = jax.lax.optimization_barrier((gum42, gum43))

    degp = _deg_kernel(dst_r, ones_ch, zeros_nd)
    h1 = _tc_call(_k1a_body, jax.ShapeDtypeStruct((N, 16), f32))(x, W1)
    d0 = degp[0, :N].reshape(N, 1)
    d1 = degp[1, :N].reshape(N, 1)

    dinv, g1 = _tc_call(_k1b_body, (
        jax.ShapeDtypeStruct((N, 1), f32),
        jax.ShapeDtypeStruct((N, 16), f32),
    ))(h1, d0, d1)
    p = _scatter16(g1, src_r, dst_r, zeros16)

    g2 = _tc_call(_k_mid_body, jax.ShapeDtypeStruct((N, 24), f32))(
        p, g1, dinv, b1, W2)
    p = _scatter24(g2, src_r, dst_r, zeros24)

    g3 = _tc_call(_k_mid_body, jax.ShapeDtypeStruct((N, 32), f32))(
        p, g2, dinv, b2, W3)
    p = _scatter32(g3, src_r, dst_r, zeros32)

    lsp, ep, lep, EL = _tc_call(_k_head_body, (
        jax.ShapeDtypeStruct((N, 8), f32),
        jax.ShapeDtypeStruct((N, 8), f32),
        jax.ShapeDtypeStruct((N, 8), f32),
        jax.ShapeDtypeStruct((N, 16), f32),
    ))(p, g3, dinv, b3, S1w, S1b, S2w, S2b, E1w, E1b, E2w, E2b)

    start_node = jnp.argmax(gum42 + lsp, axis=-1)

    idx = jnp.concatenate([start_node.astype(jnp.int32),
                           jnp.zeros((NG - N,), jnp.int32)]).reshape(NW, GSTEP, GCH)
    en1 = jnp.argmax(gum43[:N] + lep, axis=-1)
    ELs = _gather_rows(EL, idx)

    end_probs = jnp.concatenate([ep, ELs[:N, :8]], axis=0)
    en2 = jnp.argmax(gum43[N:] + ELs[:N, 8:], axis=-1)
    end_node = jnp.concatenate([en1, en2], axis=0)
    return (start_node, end_node, end_probs)
